# Initial kernel scaffold; baseline (speedup 1.0000x reference)
import jax
import jax.numpy as jnp
from jax.experimental import pallas as pl


def kernel(x, Wg, W1, b1, W2, b2, Ws1, bs1, Ws2, bs2):
    raise NotImplementedError("write your pallas kernel here")


# speedup vs baseline: 3.4969x; 3.4969x over previous
import functools

import jax
import jax.numpy as jnp
from jax.experimental import pallas as pl
from jax.experimental.pallas import tpu as pltpu

INTERPRET = False

E = 8
K = 2
BM = 256
MAX_RB = 24


def _router_kernel(x_ref, wg_ref, w0_ref, w1_ref, p0_ref, p1_ref, sp_ref):
    T = x_ref.shape[0]
    x = x_ref[...]
    wg = wg_ref[...]
    logits = jax.lax.dot_general(
        x, wg, (((1,), (1,)), ((), ())), preferred_element_type=jnp.float32)
    m = jnp.max(logits, axis=-1, keepdims=True)
    ex = jnp.exp(logits - m)
    scores = ex / jnp.sum(ex, axis=-1, keepdims=True)

    col = jax.lax.broadcasted_iota(jnp.int32, (T, E), 1).astype(jnp.float32)
    m1 = jnp.max(scores, axis=-1, keepdims=True)
    i1 = jnp.min(jnp.where(scores >= m1, col, float(E)), axis=-1, keepdims=True)
    masked = jnp.where(col == i1, -jnp.inf, scores)
    m2 = jnp.max(masked, axis=-1, keepdims=True)
    i2 = jnp.min(jnp.where(masked >= m2, col, float(E)), axis=-1, keepdims=True)

    s = m1 + m2 + 1e-20
    w0_ref[...] = m1 / s
    w1_ref[...] = m2 / s

    oh0 = (col == i1).astype(jnp.float32)
    oh1 = (col == i2).astype(jnp.float32)

    CH = 512
    r = jax.lax.broadcasted_iota(jnp.int32, (CH, CH), 0)
    c = jax.lax.broadcasted_iota(jnp.int32, (CH, CH), 1)
    lt = (r >= c).astype(jnp.float32)

    def ranks(oh, base):
        out = []
        for ci in range(T // CH):
            ohc = oh[ci * CH:(ci + 1) * CH]
            incl = jax.lax.dot_general(
                lt, ohc, (((1,), (0,)), ((), ())),
                preferred_element_type=jnp.float32)
            out.append(jnp.sum(ohc * (base + incl - 1.0), axis=-1,
                               keepdims=True))
            base = base + jnp.sum(ohc, axis=0, keepdims=True)
        return jnp.concatenate(out, axis=0), base

    zero = jnp.zeros((1, E), jnp.float32)
    rank0, base = ranks(oh0, zero)
    rank1, counts = ranks(oh1, base)

    nb = jnp.floor((counts + float(BM - 1)) / float(BM))
    er = jax.lax.broadcasted_iota(jnp.int32, (E, E), 0)
    ec = jax.lax.broadcasted_iota(jnp.int32, (E, E), 1)
    strict = (er < ec).astype(jnp.float32)
    cumb = jax.lax.dot_general(
        nb, strict, (((1,), (0,)), ((), ())),
        preferred_element_type=jnp.float32)
    cumbend = cumb + nb
    nb_total = jnp.sum(nb, axis=-1, keepdims=True)

    base0 = jnp.sum(oh0 * cumb, axis=-1, keepdims=True) * float(BM)
    base1 = jnp.sum(oh1 * cumb, axis=-1, keepdims=True) * float(BM)
    p0_ref[...] = (base0 + rank0).astype(jnp.int32)
    p1_ref[...] = (base1 + rank1).astype(jnp.int32)

    nsp = sp_ref.shape[0]
    rb = jax.lax.broadcasted_iota(jnp.int32, (nsp, E), 0).astype(jnp.float32)
    bexp = jnp.sum((rb >= cumbend).astype(jnp.float32), axis=-1, keepdims=True)
    bexp = jnp.minimum(bexp, float(E - 1))
    is_total = jax.lax.broadcasted_iota(jnp.int32, (nsp, 1), 0) == MAX_RB
    sp_ref[...] = jnp.where(is_total, nb_total, bexp).astype(jnp.int32)


def _route(x2d, Wg):
    T = x2d.shape[0]
    return pl.pallas_call(
        _router_kernel,
        out_shape=(
            jax.ShapeDtypeStruct((T, 1), jnp.float32),
            jax.ShapeDtypeStruct((T, 1), jnp.float32),
            jax.ShapeDtypeStruct((T, 1), jnp.int32),
            jax.ShapeDtypeStruct((T, 1), jnp.int32),
            jax.ShapeDtypeStruct((MAX_RB + 1, 1), jnp.int32),
        ),
        interpret=INTERPRET,
    )(x2d, Wg)


def _dispatch_kernel(p0_ref, p1_ref, x_ref, o_ref):
    T = x_ref.shape[0]

    def body(t, _):
        r0 = p0_ref[t]
        o_ref[pl.ds(r0, 1), :] = x_ref[pl.ds(t, 1), :]
        r1 = p1_ref[t]
        o_ref[pl.ds(r1, 1), :] = x_ref[pl.ds(t, 1), :]
        return ()

    jax.lax.fori_loop(0, T, body, ())


def _dispatch(x2d, p0, p1):
    T, D = x2d.shape
    return pl.pallas_call(
        _dispatch_kernel,
        out_shape=jax.ShapeDtypeStruct((MAX_RB * BM, D), jnp.float32),
        in_specs=[
            pl.BlockSpec(memory_space=pltpu.SMEM),
            pl.BlockSpec(memory_space=pltpu.SMEM),
            pl.BlockSpec(memory_space=pltpu.VMEM),
        ],
        out_specs=pl.BlockSpec(memory_space=pltpu.VMEM),
        interpret=INTERPRET,
    )(p0.reshape(T), p1.reshape(T), x2d)


def _gelu(h):
    return 0.5 * h * (1.0 + jax.lax.erf(h * 0.7071067811865476))


def _ffn_block_kernel(sp_ref, x_ref, w1_ref, b1_ref, w2_ref, b2_ref, o_ref):
    b = pl.program_id(0)

    @pl.when(b < sp_ref[MAX_RB])
    def _():
        x = x_ref[...]
        h = jax.lax.dot_general(
            x, w1_ref[...], (((1,), (1,)), ((), ())),
            preferred_element_type=jnp.float32) + b1_ref[...]
        a = _gelu(h)
        o_ref[...] = jax.lax.dot_general(
            a, w2_ref[...], (((1,), (1,)), ((), ())),
            preferred_element_type=jnp.float32) + b2_ref[...]


def _grouped_ffn(x_sorted, W1, b1, W2, b2, sp):
    D = x_sorted.shape[1]
    H = W1.shape[1]
    grid_spec = pltpu.PrefetchScalarGridSpec(
        num_scalar_prefetch=1,
        grid=(MAX_RB,),
        in_specs=[
            pl.BlockSpec((BM, D), lambda b, sp: (b, 0)),
            pl.BlockSpec((None, H, D), lambda b, sp: (sp[b], 0, 0)),
            pl.BlockSpec((None, 1, H), lambda b, sp: (sp[b], 0, 0)),
            pl.BlockSpec((None, D, H), lambda b, sp: (sp[b], 0, 0)),
            pl.BlockSpec((None, 1, D), lambda b, sp: (sp[b], 0, 0)),
        ],
        out_specs=pl.BlockSpec((BM, D), lambda b, sp: (b, 0)),
    )
    return pl.pallas_call(
        _ffn_block_kernel,
        grid_spec=grid_spec,
        out_shape=jax.ShapeDtypeStruct((MAX_RB * BM, D), jnp.float32),
        interpret=INTERPRET,
    )(sp.reshape(MAX_RB + 1), x_sorted,
      W1, b1.reshape(E, 1, H), W2, b2.reshape(E, 1, D))


def _gather_kernel(p0_ref, p1_ref, os_ref, w0_ref, w1_ref, g0_ref, g1_ref):
    T = g0_ref.shape[0]

    def body(t, _):
        r0 = p0_ref[t]
        g0_ref[pl.ds(t, 1), :] = os_ref[pl.ds(r0, 1), :]
        r1 = p1_ref[t]
        g1_ref[pl.ds(t, 1), :] = os_ref[pl.ds(r1, 1), :]
        return ()

    jax.lax.fori_loop(0, T, body, ())
    g0_ref[...] = g0_ref[...] * w0_ref[...]
    g1_ref[...] = g1_ref[...] * w1_ref[...]


def _combine_gather(out_sorted, p0, p1, w0, w1):
    D = out_sorted.shape[1]
    T = w0.shape[0]
    return pl.pallas_call(
        _gather_kernel,
        out_shape=(
            jax.ShapeDtypeStruct((T, D), jnp.float32),
            jax.ShapeDtypeStruct((T, D), jnp.float32),
        ),
        in_specs=[
            pl.BlockSpec(memory_space=pltpu.SMEM),
            pl.BlockSpec(memory_space=pltpu.SMEM),
            pl.BlockSpec(memory_space=pltpu.VMEM),
            pl.BlockSpec(memory_space=pltpu.VMEM),
            pl.BlockSpec(memory_space=pltpu.VMEM),
        ],
        out_specs=(pl.BlockSpec(memory_space=pltpu.VMEM),
                   pl.BlockSpec(memory_space=pltpu.VMEM)),
        interpret=INTERPRET,
    )(p0.reshape(T), p1.reshape(T), out_sorted, w0, w1)


def _shared_kernel(x_ref, ws1_ref, bs1_ref, ws2_ref, bs2_ref,
                   g0_ref, g1_ref, y_ref):
    x = x_ref[...]
    h = jax.lax.dot_general(
        x, ws1_ref[...], (((1,), (1,)), ((), ())),
        preferred_element_type=jnp.float32) + bs1_ref[...]
    a = _gelu(h)
    sh = jax.lax.dot_general(
        a, ws2_ref[...], (((1,), (1,)), ((), ())),
        preferred_element_type=jnp.float32) + bs2_ref[...]
    y_ref[...] = sh + g0_ref[...] + g1_ref[...]


def _shared_combine(x2d, Ws1, bs1, Ws2, bs2, g0, g1):
    T, D = x2d.shape
    H = Ws1.shape[0]
    nblk = T // BM
    return pl.pallas_call(
        _shared_kernel,
        grid=(nblk,),
        in_specs=[
            pl.BlockSpec((BM, D), lambda b: (b, 0)),
            pl.BlockSpec((H, D), lambda b: (0, 0)),
            pl.BlockSpec((1, H), lambda b: (0, 0)),
            pl.BlockSpec((D, H), lambda b: (0, 0)),
            pl.BlockSpec((1, D), lambda b: (0, 0)),
            pl.BlockSpec((BM, D), lambda b: (b, 0)),
            pl.BlockSpec((BM, D), lambda b: (b, 0)),
        ],
        out_specs=pl.BlockSpec((BM, D), lambda b: (b, 0)),
        out_shape=jax.ShapeDtypeStruct((T, D), jnp.float32),
        interpret=INTERPRET,
    )(x2d, Ws1, bs1.reshape(1, H), Ws2, bs2.reshape(1, D), g0, g1)


def kernel(x, Wg, W1, b1, W2, b2, Ws1, bs1, Ws2, bs2):
    orig_shape = x.shape
    T = orig_shape[0] * orig_shape[1]
    D = orig_shape[2]
    x2d = x.reshape(T, D)

    w0, w1, p0, p1, sp = _route(x2d, Wg)
    x_sorted = _dispatch(x2d, p0, p1)
    out_sorted = _grouped_ffn(x_sorted, W1, b1, W2, b2, sp)
    g0, g1 = _combine_gather(out_sorted, p0, p1, w0, w1)
    y = _shared_combine(x2d, Ws1, bs1, Ws2, bs2, g0, g1)

    return (jnp.float32(0.0), y.reshape(orig_shape))
